# Initial kernel scaffold; baseline (speedup 1.0000x reference)
import functools

import jax
import jax.numpy as jnp
from jax import lax
from jax.experimental import pallas as pl
from jax.experimental.pallas import tpu as pltpu
from jax.experimental.pallas import tpu_sc as plsc

N = 10000
E = 320000
D = 128

NC = 2
NS = 16
NW = NC * NS

E_PER_TILE = E // NW
CHUNK = 125
NCHUNK = E_PER_TILE // CHUNK
ROWS_PER_TILE = N // NS


def _sc_agg(h, src_r, dst_r, zeros_tile):
  mesh = plsc.VectorSubcoreMesh(
      core_axis_name="c", subcore_axis_name="s", num_cores=NC, num_subcores=NS)

  @functools.partial(
      pl.kernel,
      out_type=jax.ShapeDtypeStruct((NC, N, D), jnp.float32),
      mesh=mesh,
      scratch_types=[
          pltpu.VMEM((NCHUNK, CHUNK), jnp.int32),
          pltpu.VMEM((NCHUNK, CHUNK), jnp.int32),
          pltpu.VMEM((CHUNK, D), jnp.float32),
          pltpu.VMEM_SHARED((N, D), jnp.float32),
          pltpu.SemaphoreType.DMA,
      ],
  )
  def k(h_hbm, src_hbm, dst_hbm, z_hbm, out_hbm, src_v, dst_v, rows_v, agg_s, sem):
    c = lax.axis_index("c")
    s = lax.axis_index("s")
    wid = s * NC + c

    pltpu.sync_copy(z_hbm, agg_s.at[pl.ds(s * ROWS_PER_TILE, ROWS_PER_TILE)])
    pltpu.sync_copy(src_hbm.at[wid], src_v)
    pltpu.sync_copy(dst_hbm.at[wid], dst_v)
    plsc.subcore_barrier()

    def body(j, _):
      pltpu.async_copy(h_hbm.at[src_v.at[j]], rows_v, sem).wait()
      pltpu.sync_copy(rows_v, agg_s.at[dst_v.at[j]], add=True)
      return ()

    lax.fori_loop(0, NCHUNK, body, ())

    plsc.subcore_barrier()
    pltpu.sync_copy(
        agg_s.at[pl.ds(s * ROWS_PER_TILE, ROWS_PER_TILE)],
        out_hbm.at[c, pl.ds(s * ROWS_PER_TILE, ROWS_PER_TILE)])

  return k(h, src_r, dst_r, zeros_tile)


BN = 2000


def _mlp1_body(x_ref, p_ref, w1_ref, b1_ref, w2_ref, b2_ref, o_ref):
  z = x_ref[...] + p_ref[0] + p_ref[1]
  t = jnp.dot(z, w1_ref[...], preferred_element_type=jnp.float32,
              precision=lax.Precision.HIGHEST) + b1_ref[...]
  t = jnp.maximum(t, 0.0)
  u = jnp.dot(t, w2_ref[...], preferred_element_type=jnp.float32,
              precision=lax.Precision.HIGHEST) + b2_ref[...]
  o_ref[...] = jnp.maximum(u, 0.0)


def _mlp2_body(x_ref, p_ref, w1_ref, b1_ref, w2_ref, b2_ref, wl_ref, bl_ref,
               o_ref):
  z = x_ref[...] + p_ref[0] + p_ref[1]
  t = jnp.dot(z, w1_ref[...], preferred_element_type=jnp.float32,
              precision=lax.Precision.HIGHEST) + b1_ref[...]
  t = jnp.maximum(t, 0.0)
  u = jnp.dot(t, w2_ref[...], preferred_element_type=jnp.float32,
              precision=lax.Precision.HIGHEST) + b2_ref[...]
  u = jnp.maximum(u, 0.0)
  o_ref[...] = jnp.dot(u, wl_ref[...], preferred_element_type=jnp.float32,
                       precision=lax.Precision.HIGHEST) + bl_ref[...]


def _full(shape):
  return pl.BlockSpec(shape, lambda i: tuple(0 for _ in shape))


def _tc_mlp1(x, parts, W1, b1, W2, b2):
  return pl.pallas_call(
      _mlp1_body,
      grid=(N // BN,),
      in_specs=[
          pl.BlockSpec((BN, D), lambda i: (i, 0)),
          pl.BlockSpec((NC, BN, D), lambda i: (0, i, 0)),
          _full(W1.shape), _full((1, D)), _full(W2.shape), _full((1, D)),
      ],
      out_specs=pl.BlockSpec((BN, D), lambda i: (i, 0)),
      out_shape=jax.ShapeDtypeStruct((N, D), jnp.float32),
  )(x, parts, W1, b1.reshape(1, D), W2, b2.reshape(1, D))


def _tc_mlp2(h, parts, W1, b1, W2, b2, Wlin, blin):
  H2 = W1.shape[1]
  return pl.pallas_call(
      _mlp2_body,
      grid=(N // BN,),
      in_specs=[
          pl.BlockSpec((BN, D), lambda i: (i, 0)),
          pl.BlockSpec((NC, BN, D), lambda i: (0, i, 0)),
          _full(W1.shape), _full((1, H2)), _full(W2.shape), _full((1, H2)),
          _full(Wlin.shape), _full((1, D)),
      ],
      out_specs=pl.BlockSpec((BN, D), lambda i: (i, 0)),
      out_shape=jax.ShapeDtypeStruct((N, D), jnp.float32),
  )(h, parts, W1, b1.reshape(1, H2), W2, b2.reshape(1, H2),
    Wlin, blin.reshape(1, D))


def kernel(x, edge_index, W1a, b1a, W2a, b2a, W1b, b1b, W2b, b2b, Wlin, blin):
  ei = edge_index.astype(jnp.int32)
  src_r = ei[0].reshape(NW, NCHUNK, CHUNK)
  dst_r = ei[1].reshape(NW, NCHUNK, CHUNK)
  zeros_tile = jnp.zeros((ROWS_PER_TILE, D), jnp.float32)

  p1 = _sc_agg(x, src_r, dst_r, zeros_tile)
  h1 = _tc_mlp1(x, p1, W1a, b1a, W2a, b2a)
  p2 = _sc_agg(h1, src_r, dst_r, zeros_tile)
  out = _tc_mlp2(h1, p2, W1b, b1b, W2b, b2b, Wlin, blin)
  return out


# speedup vs baseline: 7.5869x; 7.5869x over previous
import functools

import jax
import jax.numpy as jnp
from jax import lax
from jax.experimental import pallas as pl
from jax.experimental.pallas import tpu as pltpu
from jax.experimental.pallas import tpu_sc as plsc

N = 10000
E = 320000
D = 128

NC = 2
NS = 16
NW = NC * NS

E_PER_TILE = E // NW
CHUNK = 125
NCHUNK = E_PER_TILE // CHUNK
N_PAD = 10240
ROWS_PER_TILE = N_PAD // NS


def _sc_agg(h, src_r, dst_r, zeros_tile):
  mesh = plsc.VectorSubcoreMesh(
      core_axis_name="c", subcore_axis_name="s", num_cores=NC, num_subcores=NS)

  @functools.partial(
      pl.kernel,
      out_type=jax.ShapeDtypeStruct((NC, N_PAD, D), jnp.float32),
      mesh=mesh,
      scratch_types=[
          pltpu.VMEM((NCHUNK, CHUNK), jnp.int32),
          pltpu.VMEM((NCHUNK, CHUNK), jnp.int32),
          pltpu.VMEM((CHUNK, D), jnp.float32),
          pltpu.VMEM_SHARED((N_PAD, D), jnp.float32),
          pltpu.SemaphoreType.DMA,
      ],
  )
  def k(h_hbm, src_hbm, dst_hbm, z_hbm, out_hbm, src_v, dst_v, rows_v, agg_s, sem):
    c = lax.axis_index("c")
    s = lax.axis_index("s")
    wid = s * NC + c

    pltpu.sync_copy(z_hbm, agg_s.at[pl.ds(s * ROWS_PER_TILE, ROWS_PER_TILE)])
    pltpu.sync_copy(src_hbm.at[wid], src_v)
    pltpu.sync_copy(dst_hbm.at[wid], dst_v)
    plsc.subcore_barrier()

    def body(j, _):
      pltpu.async_copy(h_hbm.at[src_v.at[j]], rows_v, sem).wait()
      pltpu.sync_copy(rows_v, agg_s.at[dst_v.at[j]], add=True)
      return ()

    lax.fori_loop(0, NCHUNK, body, ())

    plsc.subcore_barrier()
    pltpu.sync_copy(
        agg_s.at[pl.ds(s * ROWS_PER_TILE, ROWS_PER_TILE)],
        out_hbm.at[c, pl.ds(s * ROWS_PER_TILE, ROWS_PER_TILE)])

  return k(h, src_r, dst_r, zeros_tile)


BN = 2000


def _mlp1_body(x_ref, p_ref, w1_ref, b1_ref, w2_ref, b2_ref, o_ref):
  z = x_ref[...] + p_ref[0] + p_ref[1]
  t = jnp.dot(z, w1_ref[...], preferred_element_type=jnp.float32,
              precision=lax.Precision.HIGHEST) + b1_ref[...]
  t = jnp.maximum(t, 0.0)
  u = jnp.dot(t, w2_ref[...], preferred_element_type=jnp.float32,
              precision=lax.Precision.HIGHEST) + b2_ref[...]
  o_ref[...] = jnp.maximum(u, 0.0)


def _mlp2_body(x_ref, p_ref, w1_ref, b1_ref, w2_ref, b2_ref, wl_ref, bl_ref,
               o_ref):
  z = x_ref[...] + p_ref[0] + p_ref[1]
  t = jnp.dot(z, w1_ref[...], preferred_element_type=jnp.float32,
              precision=lax.Precision.HIGHEST) + b1_ref[...]
  t = jnp.maximum(t, 0.0)
  u = jnp.dot(t, w2_ref[...], preferred_element_type=jnp.float32,
              precision=lax.Precision.HIGHEST) + b2_ref[...]
  u = jnp.maximum(u, 0.0)
  o_ref[...] = jnp.dot(u, wl_ref[...], preferred_element_type=jnp.float32,
                       precision=lax.Precision.HIGHEST) + bl_ref[...]


def _full(shape):
  return pl.BlockSpec(shape, lambda i: tuple(0 for _ in shape))


def _tc_mlp1(x, parts, W1, b1, W2, b2):
  return pl.pallas_call(
      _mlp1_body,
      grid=(N // BN,),
      in_specs=[
          pl.BlockSpec((BN, D), lambda i: (i, 0)),
          pl.BlockSpec((NC, BN, D), lambda i: (0, i, 0)),
          _full(W1.shape), _full((1, D)), _full(W2.shape), _full((1, D)),
      ],
      out_specs=pl.BlockSpec((BN, D), lambda i: (i, 0)),
      out_shape=jax.ShapeDtypeStruct((N, D), jnp.float32),
  )(x, parts, W1, b1.reshape(1, D), W2, b2.reshape(1, D))


def _tc_mlp2(h, parts, W1, b1, W2, b2, Wlin, blin):
  H2 = W1.shape[1]
  return pl.pallas_call(
      _mlp2_body,
      grid=(N // BN,),
      in_specs=[
          pl.BlockSpec((BN, D), lambda i: (i, 0)),
          pl.BlockSpec((NC, BN, D), lambda i: (0, i, 0)),
          _full(W1.shape), _full((1, H2)), _full(W2.shape), _full((1, H2)),
          _full(Wlin.shape), _full((1, D)),
      ],
      out_specs=pl.BlockSpec((BN, D), lambda i: (i, 0)),
      out_shape=jax.ShapeDtypeStruct((N, D), jnp.float32),
  )(h, parts, W1, b1.reshape(1, H2), W2, b2.reshape(1, H2),
    Wlin, blin.reshape(1, D))


def kernel(x, edge_index, W1a, b1a, W2a, b2a, W1b, b1b, W2b, b2b, Wlin, blin):
  ei = edge_index.astype(jnp.int32)
  src_r = ei[0].reshape(NW, NCHUNK, CHUNK)
  dst_r = ei[1].reshape(NW, NCHUNK, CHUNK)
  zeros_tile = jnp.zeros((ROWS_PER_TILE, D), jnp.float32)

  p1 = _sc_agg(x, src_r, dst_r, zeros_tile)
  h1 = _tc_mlp1(x, p1, W1a, b1a, W2a, b2a)
  p2 = _sc_agg(h1, src_r, dst_r, zeros_tile)
  out = _tc_mlp2(h1, p2, W1b, b1b, W2b, b2b, Wlin, blin)
  return out


# speedup vs baseline: 7.5881x; 1.0002x over previous
import functools

import jax
import jax.numpy as jnp
from jax import lax
from jax.experimental import pallas as pl
from jax.experimental.pallas import tpu as pltpu
from jax.experimental.pallas import tpu_sc as plsc

N = 10000
E = 320000
D = 128

NC = 2
NS = 16
NW = NC * NS

E_PER_TILE = E // NW
CHUNK = 125
NCHUNK = E_PER_TILE // CHUNK
N_PAD = 10240
ROWS_PER_TILE = N_PAD // NS
NBUF = 4


def _sc_agg(h, src_r, dst_r, zeros_tile):
  mesh = plsc.VectorSubcoreMesh(
      core_axis_name="c", subcore_axis_name="s", num_cores=NC, num_subcores=NS)

  @functools.partial(
      pl.kernel,
      out_type=jax.ShapeDtypeStruct((NC, N_PAD, D), jnp.float32),
      mesh=mesh,
      scratch_types=[
          pltpu.VMEM((NCHUNK, CHUNK), jnp.int32),
          pltpu.VMEM((NCHUNK, CHUNK), jnp.int32),
          [pltpu.VMEM((CHUNK, D), jnp.float32) for _ in range(NBUF)],
          pltpu.VMEM_SHARED((N_PAD, D), jnp.float32),
          [pltpu.SemaphoreType.DMA for _ in range(NBUF)],
      ],
  )
  def k(h_hbm, src_hbm, dst_hbm, z_hbm, out_hbm, src_v, dst_v, bufs, agg_s, sems):
    c = lax.axis_index("c")
    s = lax.axis_index("s")
    wid = s * NC + c

    pltpu.sync_copy(z_hbm, agg_s.at[pl.ds(s * ROWS_PER_TILE, ROWS_PER_TILE)])
    pltpu.sync_copy(src_hbm.at[wid], src_v)
    pltpu.sync_copy(dst_hbm.at[wid], dst_v)
    plsc.subcore_barrier()

    def body(j, _):
      pltpu.async_copy(h_hbm.at[src_v.at[j]], bufs[0], sems[0]).wait()
      pltpu.sync_copy(bufs[0], agg_s.at[dst_v.at[j]], add=True)
      return ()

    lax.fori_loop(0, NCHUNK, body, ())

    plsc.subcore_barrier()
    pltpu.sync_copy(
        agg_s.at[pl.ds(s * ROWS_PER_TILE, ROWS_PER_TILE)],
        out_hbm.at[c, pl.ds(s * ROWS_PER_TILE, ROWS_PER_TILE)])

  return k(h, src_r, dst_r, zeros_tile)


BN = 2000


def _mlp1_body(x_ref, p_ref, w1_ref, b1_ref, w2_ref, b2_ref, o_ref):
  z = x_ref[...] + p_ref[0] + p_ref[1]
  t = jnp.dot(z, w1_ref[...], preferred_element_type=jnp.float32,
              precision=lax.Precision.HIGHEST) + b1_ref[...]
  t = jnp.maximum(t, 0.0)
  u = jnp.dot(t, w2_ref[...], preferred_element_type=jnp.float32,
              precision=lax.Precision.HIGHEST) + b2_ref[...]
  o_ref[...] = jnp.maximum(u, 0.0)


def _mlp2_body(x_ref, p_ref, w1_ref, b1_ref, w2_ref, b2_ref, wl_ref, bl_ref,
               o_ref):
  z = x_ref[...] + p_ref[0] + p_ref[1]
  t = jnp.dot(z, w1_ref[...], preferred_element_type=jnp.float32,
              precision=lax.Precision.HIGHEST) + b1_ref[...]
  t = jnp.maximum(t, 0.0)
  u = jnp.dot(t, w2_ref[...], preferred_element_type=jnp.float32,
              precision=lax.Precision.HIGHEST) + b2_ref[...]
  u = jnp.maximum(u, 0.0)
  o_ref[...] = jnp.dot(u, wl_ref[...], preferred_element_type=jnp.float32,
                       precision=lax.Precision.HIGHEST) + bl_ref[...]


def _full(shape):
  return pl.BlockSpec(shape, lambda i: tuple(0 for _ in shape))


def _tc_mlp1(x, parts, W1, b1, W2, b2):
  return pl.pallas_call(
      _mlp1_body,
      grid=(N // BN,),
      in_specs=[
          pl.BlockSpec((BN, D), lambda i: (i, 0)),
          pl.BlockSpec((NC, BN, D), lambda i: (0, i, 0)),
          _full(W1.shape), _full((1, D)), _full(W2.shape), _full((1, D)),
      ],
      out_specs=pl.BlockSpec((BN, D), lambda i: (i, 0)),
      out_shape=jax.ShapeDtypeStruct((N, D), jnp.float32),
  )(x, parts, W1, b1.reshape(1, D), W2, b2.reshape(1, D))


def _tc_mlp2(h, parts, W1, b1, W2, b2, Wlin, blin):
  H2 = W1.shape[1]
  return pl.pallas_call(
      _mlp2_body,
      grid=(N // BN,),
      in_specs=[
          pl.BlockSpec((BN, D), lambda i: (i, 0)),
          pl.BlockSpec((NC, BN, D), lambda i: (0, i, 0)),
          _full(W1.shape), _full((1, H2)), _full(W2.shape), _full((1, H2)),
          _full(Wlin.shape), _full((1, D)),
      ],
      out_specs=pl.BlockSpec((BN, D), lambda i: (i, 0)),
      out_shape=jax.ShapeDtypeStruct((N, D), jnp.float32),
  )(h, parts, W1, b1.reshape(1, H2), W2, b2.reshape(1, H2),
    Wlin, blin.reshape(1, D))


def kernel(x, edge_index, W1a, b1a, W2a, b2a, W1b, b1b, W2b, b2b, Wlin, blin):
  ei = edge_index.astype(jnp.int32)
  src_r = ei[0].reshape(NW, NCHUNK, CHUNK)
  dst_r = ei[1].reshape(NW, NCHUNK, CHUNK)
  zeros_tile = jnp.zeros((ROWS_PER_TILE, D), jnp.float32)

  p1 = _sc_agg(x, src_r, dst_r, zeros_tile)
  h1 = _tc_mlp1(x, p1, W1a, b1a, W2a, b2a)
  p2 = _sc_agg(h1, src_r, dst_r, zeros_tile)
  out = _tc_mlp2(h1, p2, W1b, b1b, W2b, b2b, Wlin, blin)
  return out


# speedup vs baseline: 10.9445x; 1.4423x over previous
import functools

import jax
import jax.numpy as jnp
from jax import lax
from jax.experimental import pallas as pl
from jax.experimental.pallas import tpu as pltpu
from jax.experimental.pallas import tpu_sc as plsc

N = 10000
E = 320000
D = 128

NC = 2
NS = 16
NW = NC * NS

E_PER_TILE = E // NW
CHUNK = 125
NCHUNK = E_PER_TILE // CHUNK
N_PAD = 10240
ROWS_PER_TILE = N_PAD // NS
W = 8
NWIN = NCHUNK // W


def _sc_agg(h, src_r, dst_r, zeros_tile):
  mesh = plsc.VectorSubcoreMesh(
      core_axis_name="c", subcore_axis_name="s", num_cores=NC, num_subcores=NS)

  @functools.partial(
      pl.kernel,
      out_type=jax.ShapeDtypeStruct((NC, N_PAD, D), jnp.float32),
      mesh=mesh,
      scratch_types=[
          pltpu.VMEM((NCHUNK, CHUNK), jnp.int32),
          [pltpu.VMEM((W, CHUNK), jnp.int32) for _ in range(2)],
          [pltpu.VMEM((CHUNK, D), jnp.float32) for _ in range(2)],
          pltpu.VMEM_SHARED((N_PAD, D), jnp.float32),
          [pltpu.SemaphoreType.DMA for _ in range(2)],
          [pltpu.SemaphoreType.DMA for _ in range(2)],
      ],
  )
  def k(h_hbm, src_hbm, dst_hbm, z_hbm, out_hbm, src_v, dws, bufs, agg_s,
        dsems, gsems):
    c = lax.axis_index("c")
    s = lax.axis_index("s")
    wid = s * NC + c

    pltpu.sync_copy(z_hbm, agg_s.at[pl.ds(s * ROWS_PER_TILE, ROWS_PER_TILE)])
    pltpu.sync_copy(src_hbm.at[wid], src_v)
    plsc.subcore_barrier()

    def fetch_dst(w, slot):
      pltpu.async_copy(
          dst_hbm.at[wid, pl.ds(w * W, W)], dws[slot], dsems[slot])

    def fire_gather(j, b):
      pltpu.async_copy(h_hbm.at[src_v.at[j]], bufs[b], gsems[b])

    def process_window(w, slot, fire_ks):
      pltpu.make_async_copy(
          dst_hbm.at[wid, pl.ds(0, W)], dws[slot], dsems[slot]).wait()
      for k_ in range(W):
        j = w * W + k_
        b = k_ % 2
        pltpu.make_async_copy(
            h_hbm.at[src_v.at[j]], bufs[b], gsems[b]).wait()
        pltpu.sync_copy(bufs[b], agg_s.at[dws[slot].at[k_]], add=True)
        if k_ in fire_ks:
          fire_gather(j + 2, b)

    ALL = tuple(range(W))
    fetch_dst(0, 0)
    fire_gather(0, 0)
    fire_gather(1, 1)

    def body2(t, _):
      fetch_dst(2 * t + 1, 1)
      process_window(2 * t, 0, ALL)
      fetch_dst(2 * t + 2, 0)
      process_window(2 * t + 1, 1, ALL)
      return ()

    lax.fori_loop(0, NWIN // 2 - 1, body2, ())
    fetch_dst(NWIN - 1, 1)
    process_window(NWIN - 2, 0, ALL)
    process_window(NWIN - 1, 1, tuple(range(W - 2)))

    plsc.subcore_barrier()
    pltpu.sync_copy(
        agg_s.at[pl.ds(s * ROWS_PER_TILE, ROWS_PER_TILE)],
        out_hbm.at[c, pl.ds(s * ROWS_PER_TILE, ROWS_PER_TILE)])

  return k(h, src_r, dst_r, zeros_tile)


BN = 2000


def _mlp1_body(x_ref, p_ref, w1_ref, b1_ref, w2_ref, b2_ref, o_ref):
  z = x_ref[...] + p_ref[0] + p_ref[1]
  t = jnp.dot(z, w1_ref[...], preferred_element_type=jnp.float32,
              precision=lax.Precision.HIGHEST) + b1_ref[...]
  t = jnp.maximum(t, 0.0)
  u = jnp.dot(t, w2_ref[...], preferred_element_type=jnp.float32,
              precision=lax.Precision.HIGHEST) + b2_ref[...]
  o_ref[...] = jnp.maximum(u, 0.0)


def _mlp2_body(x_ref, p_ref, w1_ref, b1_ref, w2_ref, b2_ref, wl_ref, bl_ref,
               o_ref):
  z = x_ref[...] + p_ref[0] + p_ref[1]
  t = jnp.dot(z, w1_ref[...], preferred_element_type=jnp.float32,
              precision=lax.Precision.HIGHEST) + b1_ref[...]
  t = jnp.maximum(t, 0.0)
  u = jnp.dot(t, w2_ref[...], preferred_element_type=jnp.float32,
              precision=lax.Precision.HIGHEST) + b2_ref[...]
  u = jnp.maximum(u, 0.0)
  o_ref[...] = jnp.dot(u, wl_ref[...], preferred_element_type=jnp.float32,
                       precision=lax.Precision.HIGHEST) + bl_ref[...]


def _full(shape):
  return pl.BlockSpec(shape, lambda i: tuple(0 for _ in shape))


def _tc_mlp1(x, parts, W1, b1, W2, b2):
  return pl.pallas_call(
      _mlp1_body,
      grid=(N // BN,),
      in_specs=[
          pl.BlockSpec((BN, D), lambda i: (i, 0)),
          pl.BlockSpec((NC, BN, D), lambda i: (0, i, 0)),
          _full(W1.shape), _full((1, D)), _full(W2.shape), _full((1, D)),
      ],
      out_specs=pl.BlockSpec((BN, D), lambda i: (i, 0)),
      out_shape=jax.ShapeDtypeStruct((N, D), jnp.float32),
  )(x, parts, W1, b1.reshape(1, D), W2, b2.reshape(1, D))


def _tc_mlp2(h, parts, W1, b1, W2, b2, Wlin, blin):
  H2 = W1.shape[1]
  return pl.pallas_call(
      _mlp2_body,
      grid=(N // BN,),
      in_specs=[
          pl.BlockSpec((BN, D), lambda i: (i, 0)),
          pl.BlockSpec((NC, BN, D), lambda i: (0, i, 0)),
          _full(W1.shape), _full((1, H2)), _full(W2.shape), _full((1, H2)),
          _full(Wlin.shape), _full((1, D)),
      ],
      out_specs=pl.BlockSpec((BN, D), lambda i: (i, 0)),
      out_shape=jax.ShapeDtypeStruct((N, D), jnp.float32),
  )(h, parts, W1, b1.reshape(1, H2), W2, b2.reshape(1, H2),
    Wlin, blin.reshape(1, D))


def kernel(x, edge_index, W1a, b1a, W2a, b2a, W1b, b1b, W2b, b2b, Wlin, blin):
  ei = edge_index.astype(jnp.int32)
  src_r = ei[0].reshape(NW, NCHUNK, CHUNK)
  dst_r = ei[1].reshape(NW, NCHUNK, CHUNK)
  zeros_tile = jnp.zeros((ROWS_PER_TILE, D), jnp.float32)

  p1 = _sc_agg(x, src_r, dst_r, zeros_tile)
  h1 = _tc_mlp1(x, p1, W1a, b1a, W2a, b2a)
  p2 = _sc_agg(h1, src_r, dst_r, zeros_tile)
  out = _tc_mlp2(h1, p2, W1b, b1b, W2b, b2b, Wlin, blin)
  return out


# speedup vs baseline: 12.5815x; 1.1496x over previous
import functools

import jax
import jax.numpy as jnp
from jax import lax
from jax.experimental import pallas as pl
from jax.experimental.pallas import tpu as pltpu
from jax.experimental.pallas import tpu_sc as plsc

N = 10000
E = 320000
D = 128

NC = 2
NS = 16
NW = NC * NS

E_PER_TILE = E // NW
CHUNK = 125
NCHUNK = E_PER_TILE // CHUNK
N_PAD = 10240
ROWS_PER_TILE = N_PAD // NS
W = 8
NWIN = NCHUNK // W


def _sc_agg(h, src_r, dst_r, zeros_tile):
  mesh = plsc.VectorSubcoreMesh(
      core_axis_name="c", subcore_axis_name="s", num_cores=NC, num_subcores=NS)

  @functools.partial(
      pl.kernel,
      out_type=jax.ShapeDtypeStruct((NC, N_PAD, D), jnp.float32),
      mesh=mesh,
      scratch_types=[
          pltpu.VMEM((NCHUNK, CHUNK), jnp.int32),
          [pltpu.VMEM((W, CHUNK), jnp.int32) for _ in range(2)],
          [pltpu.VMEM((CHUNK, D), jnp.float32) for _ in range(2)],
          pltpu.VMEM_SHARED((N_PAD, D), jnp.float32),
          [pltpu.SemaphoreType.DMA for _ in range(2)],
          [pltpu.SemaphoreType.DMA for _ in range(2)],
      ],
  )
  def k(h_hbm, src_hbm, dst_hbm, z_hbm, out_hbm, src_v, dws, bufs, agg_s,
        dsems, gsems):
    c = lax.axis_index("c")
    s = lax.axis_index("s")
    wid = s * NC + c

    pltpu.sync_copy(z_hbm, agg_s.at[pl.ds(s * ROWS_PER_TILE, ROWS_PER_TILE)])
    pltpu.sync_copy(src_hbm.at[wid], src_v)
    plsc.subcore_barrier()

    def fetch_dst(w, slot):
      pltpu.async_copy(
          dst_hbm.at[wid, pl.ds(w * W, W)], dws[slot], dsems[slot])

    def fire_gather(j, b):
      pltpu.async_copy(h_hbm.at[src_v.at[j]], bufs[b], gsems[b])

    def process_window(w, slot, fire_ks):
      pltpu.make_async_copy(
          dst_hbm.at[wid, pl.ds(0, W)], dws[slot], dsems[slot]).wait()
      for k_ in range(W):
        j = w * W + k_
        b = k_ % 2
        pltpu.make_async_copy(
            h_hbm.at[src_v.at[j]], bufs[b], gsems[b]).wait()
        pltpu.sync_copy(bufs[b], agg_s.at[dws[slot].at[k_]], add=True)
        if k_ in fire_ks:
          fire_gather(j + 2, b)

    ALL = tuple(range(W))
    fetch_dst(0, 0)
    fire_gather(0, 0)
    fire_gather(1, 1)

    def body2(t, _):
      fetch_dst(2 * t + 1, 1)
      process_window(2 * t, 0, ALL)
      fetch_dst(2 * t + 2, 0)
      process_window(2 * t + 1, 1, ALL)
      return ()

    lax.fori_loop(0, NWIN // 2 - 1, body2, ())
    fetch_dst(NWIN - 1, 1)
    process_window(NWIN - 2, 0, ALL)
    process_window(NWIN - 1, 1, tuple(range(W - 2)))

    plsc.subcore_barrier()
    pltpu.sync_copy(
        agg_s.at[pl.ds(s * ROWS_PER_TILE, ROWS_PER_TILE)],
        out_hbm.at[c, pl.ds(s * ROWS_PER_TILE, ROWS_PER_TILE)])

  return k(h, src_r, dst_r, zeros_tile)


BN = 2000


def _mlp1_body(x_ref, p_ref, w1_ref, b1_ref, w2_ref, b2_ref, o_ref):
  z = x_ref[...] + p_ref[0] + p_ref[1]
  t = jnp.dot(z, w1_ref[...], preferred_element_type=jnp.float32) + b1_ref[...]
  t = jnp.maximum(t, 0.0)
  u = jnp.dot(t, w2_ref[...], preferred_element_type=jnp.float32) + b2_ref[...]
  o_ref[...] = jnp.maximum(u, 0.0)


def _mlp2_body(x_ref, p_ref, w1_ref, b1_ref, w2_ref, b2_ref, wl_ref, bl_ref,
               o_ref):
  z = x_ref[...] + p_ref[0] + p_ref[1]
  t = jnp.dot(z, w1_ref[...], preferred_element_type=jnp.float32) + b1_ref[...]
  t = jnp.maximum(t, 0.0)
  u = jnp.dot(t, w2_ref[...], preferred_element_type=jnp.float32) + b2_ref[...]
  u = jnp.maximum(u, 0.0)
  o_ref[...] = jnp.dot(u, wl_ref[...], preferred_element_type=jnp.float32) + bl_ref[...]


def _full(shape):
  return pl.BlockSpec(shape, lambda i: tuple(0 for _ in shape))


def _tc_mlp1(x, parts, W1, b1, W2, b2):
  return pl.pallas_call(
      _mlp1_body,
      grid=(N // BN,),
      in_specs=[
          pl.BlockSpec((BN, D), lambda i: (i, 0)),
          pl.BlockSpec((NC, BN, D), lambda i: (0, i, 0)),
          _full(W1.shape), _full((1, D)), _full(W2.shape), _full((1, D)),
      ],
      out_specs=pl.BlockSpec((BN, D), lambda i: (i, 0)),
      out_shape=jax.ShapeDtypeStruct((N, D), jnp.float32),
  )(x, parts, W1, b1.reshape(1, D), W2, b2.reshape(1, D))


def _tc_mlp2(h, parts, W1, b1, W2, b2, Wlin, blin):
  H2 = W1.shape[1]
  return pl.pallas_call(
      _mlp2_body,
      grid=(N // BN,),
      in_specs=[
          pl.BlockSpec((BN, D), lambda i: (i, 0)),
          pl.BlockSpec((NC, BN, D), lambda i: (0, i, 0)),
          _full(W1.shape), _full((1, H2)), _full(W2.shape), _full((1, H2)),
          _full(Wlin.shape), _full((1, D)),
      ],
      out_specs=pl.BlockSpec((BN, D), lambda i: (i, 0)),
      out_shape=jax.ShapeDtypeStruct((N, D), jnp.float32),
  )(h, parts, W1, b1.reshape(1, H2), W2, b2.reshape(1, H2),
    Wlin, blin.reshape(1, D))


def kernel(x, edge_index, W1a, b1a, W2a, b2a, W1b, b1b, W2b, b2b, Wlin, blin):
  ei = edge_index.astype(jnp.int32)
  src_r = ei[0].reshape(NW, NCHUNK, CHUNK)
  dst_r = ei[1].reshape(NW, NCHUNK, CHUNK)
  zeros_tile = jnp.zeros((ROWS_PER_TILE, D), jnp.float32)

  p1 = _sc_agg(x, src_r, dst_r, zeros_tile)
  h1 = _tc_mlp1(x, p1, W1a, b1a, W2a, b2a)
  p2 = _sc_agg(h1, src_r, dst_r, zeros_tile)
  out = _tc_mlp2(h1, p2, W1b, b1b, W2b, b2b, Wlin, blin)
  return out


# speedup vs baseline: 13.1332x; 1.0439x over previous
import functools

import jax
import jax.numpy as jnp
from jax import lax
from jax.experimental import pallas as pl
from jax.experimental.pallas import tpu as pltpu
from jax.experimental.pallas import tpu_sc as plsc

N = 10000
E = 320000
D = 128

NC = 2
NS = 16
NW = NC * NS

E_PER_TILE = E // NW
CHUNK = 125
NCHUNK = E_PER_TILE // CHUNK
N_PAD = 10240
ROWS_PER_TILE = N_PAD // NS
W = 8
NWIN = NCHUNK // W


def _sc_agg(h, ei4, zeros_tile):
  mesh = plsc.VectorSubcoreMesh(
      core_axis_name="c", subcore_axis_name="s", num_cores=NC, num_subcores=NS)

  @functools.partial(
      pl.kernel,
      out_type=jax.ShapeDtypeStruct((NC, N_PAD, D), jnp.float32),
      mesh=mesh,
      scratch_types=[
          pltpu.VMEM((NCHUNK, CHUNK), jnp.int32),
          [pltpu.VMEM((W, CHUNK), jnp.int32) for _ in range(2)],
          [pltpu.VMEM((CHUNK, D), jnp.float32) for _ in range(2)],
          pltpu.VMEM_SHARED((N_PAD, D), jnp.float32),
          [pltpu.SemaphoreType.DMA for _ in range(2)],
          [pltpu.SemaphoreType.DMA for _ in range(2)],
      ],
  )
  def k(h_hbm, ei_hbm, z_hbm, out_hbm, src_v, dws, bufs, agg_s,
        dsems, gsems):
    c = lax.axis_index("c")
    s = lax.axis_index("s")
    wid = s * NC + c

    def fetch_dst(w, slot):
      pltpu.async_copy(
          ei_hbm.at[1, wid, pl.ds(w * W, W)], dws[slot], dsems[slot])

    def fire_gather(j, b):
      pltpu.async_copy(h_hbm.at[src_v.at[j]], bufs[b], gsems[b])

    pltpu.sync_copy(ei_hbm.at[0, wid], src_v)
    fetch_dst(0, 0)
    fire_gather(0, 0)
    fire_gather(1, 1)
    pltpu.sync_copy(z_hbm, agg_s.at[pl.ds(s * ROWS_PER_TILE, ROWS_PER_TILE)])
    plsc.subcore_barrier()

    def process_window(w, slot, fire_ks):
      pltpu.make_async_copy(
          ei_hbm.at[1, wid, pl.ds(0, W)], dws[slot], dsems[slot]).wait()
      for k_ in range(W):
        j = w * W + k_
        b = k_ % 2
        pltpu.make_async_copy(
            h_hbm.at[src_v.at[j]], bufs[b], gsems[b]).wait()
        pltpu.sync_copy(bufs[b], agg_s.at[dws[slot].at[k_]], add=True)
        if k_ in fire_ks:
          fire_gather(j + 2, b)

    ALL = tuple(range(W))

    def body2(t, _):
      fetch_dst(2 * t + 1, 1)
      process_window(2 * t, 0, ALL)
      fetch_dst(2 * t + 2, 0)
      process_window(2 * t + 1, 1, ALL)
      return ()

    lax.fori_loop(0, NWIN // 2 - 1, body2, ())
    fetch_dst(NWIN - 1, 1)
    process_window(NWIN - 2, 0, ALL)
    process_window(NWIN - 1, 1, tuple(range(W - 2)))

    plsc.subcore_barrier()
    pltpu.sync_copy(
        agg_s.at[pl.ds(s * ROWS_PER_TILE, ROWS_PER_TILE)],
        out_hbm.at[c, pl.ds(s * ROWS_PER_TILE, ROWS_PER_TILE)])

  return k(h, ei4, zeros_tile)


BN = 2000


def _mlp1_body(x_ref, p_ref, w1_ref, b1_ref, w2_ref, b2_ref, o_ref):
  z = x_ref[...] + p_ref[0] + p_ref[1]
  t = jnp.dot(z, w1_ref[...], preferred_element_type=jnp.float32) + b1_ref[...]
  t = jnp.maximum(t, 0.0)
  u = jnp.dot(t, w2_ref[...], preferred_element_type=jnp.float32) + b2_ref[...]
  o_ref[...] = jnp.maximum(u, 0.0)


def _mlp2_body(x_ref, p_ref, w1_ref, b1_ref, w2_ref, b2_ref, wl_ref, bl_ref,
               o_ref):
  z = x_ref[...] + p_ref[0] + p_ref[1]
  t = jnp.dot(z, w1_ref[...], preferred_element_type=jnp.float32) + b1_ref[...]
  t = jnp.maximum(t, 0.0)
  u = jnp.dot(t, w2_ref[...], preferred_element_type=jnp.float32) + b2_ref[...]
  u = jnp.maximum(u, 0.0)
  o_ref[...] = jnp.dot(u, wl_ref[...], preferred_element_type=jnp.float32) + bl_ref[...]


def _full(shape):
  return pl.BlockSpec(shape, lambda i: tuple(0 for _ in shape))


def _tc_mlp1(x, parts, W1, b1, W2, b2):
  return pl.pallas_call(
      _mlp1_body,
      grid=(N // BN,),
      in_specs=[
          pl.BlockSpec((BN, D), lambda i: (i, 0)),
          pl.BlockSpec((NC, BN, D), lambda i: (0, i, 0)),
          _full(W1.shape), _full((1, D)), _full(W2.shape), _full((1, D)),
      ],
      out_specs=pl.BlockSpec((BN, D), lambda i: (i, 0)),
      out_shape=jax.ShapeDtypeStruct((N, D), jnp.float32),
  )(x, parts, W1, b1.reshape(1, D), W2, b2.reshape(1, D))


def _tc_mlp2(h, parts, W1, b1, W2, b2, Wlin, blin):
  H2 = W1.shape[1]
  return pl.pallas_call(
      _mlp2_body,
      grid=(N // BN,),
      in_specs=[
          pl.BlockSpec((BN, D), lambda i: (i, 0)),
          pl.BlockSpec((NC, BN, D), lambda i: (0, i, 0)),
          _full(W1.shape), _full((1, H2)), _full(W2.shape), _full((1, H2)),
          _full(Wlin.shape), _full((1, D)),
      ],
      out_specs=pl.BlockSpec((BN, D), lambda i: (i, 0)),
      out_shape=jax.ShapeDtypeStruct((N, D), jnp.float32),
  )(h, parts, W1, b1.reshape(1, H2), W2, b2.reshape(1, H2),
    Wlin, blin.reshape(1, D))


def kernel(x, edge_index, W1a, b1a, W2a, b2a, W1b, b1b, W2b, b2b, Wlin, blin):
  ei4 = edge_index.astype(jnp.int32).reshape(2, NW, NCHUNK, CHUNK)
  zeros_tile = jnp.zeros((ROWS_PER_TILE, D), jnp.float32)

  p1 = _sc_agg(x, ei4, zeros_tile)
  h1 = _tc_mlp1(x, p1, W1a, b1a, W2a, b2a)
  p2 = _sc_agg(h1, ei4, zeros_tile)
  out = _tc_mlp2(h1, p2, W1b, b1b, W2b, b2b, Wlin, blin)
  return out


# speedup vs baseline: 13.1771x; 1.0033x over previous
import functools

import jax
import jax.numpy as jnp
from jax import lax
from jax.experimental import pallas as pl
from jax.experimental.pallas import tpu as pltpu
from jax.experimental.pallas import tpu_sc as plsc

N = 10000
E = 320000
D = 128

NC = 2
NS = 16
NW = NC * NS

E_PER_TILE = E // NW
CHUNK = 125
NCHUNK = E_PER_TILE // CHUNK
N_PAD = 10240
ROWS_PER_TILE = N_PAD // NS
W = 8
NWIN = NCHUNK // W


def _sc_agg(h, ei4, zeros_tile):
  mesh = plsc.VectorSubcoreMesh(
      core_axis_name="c", subcore_axis_name="s", num_cores=NC, num_subcores=NS)

  @functools.partial(
      pl.kernel,
      out_type=jax.ShapeDtypeStruct((NC, N_PAD, D), jnp.float32),
      mesh=mesh,
      scratch_types=[
          pltpu.VMEM((NCHUNK, CHUNK), jnp.int32),
          [pltpu.VMEM((W, CHUNK), jnp.int32) for _ in range(2)],
          [pltpu.VMEM((CHUNK, D), jnp.float32) for _ in range(2)],
          pltpu.VMEM_SHARED((N_PAD, D), jnp.float32),
          [pltpu.SemaphoreType.DMA for _ in range(2)],
          [pltpu.SemaphoreType.DMA for _ in range(2)],
      ],
  )
  def k(h_hbm, ei_hbm, z_hbm, out_hbm, src_v, dws, bufs, agg_s,
        dsems, gsems):
    c = lax.axis_index("c")
    s = lax.axis_index("s")
    wid = s * NC + c

    def fetch_dst(w, slot):
      pltpu.async_copy(
          ei_hbm.at[1, wid, pl.ds(w * W, W)], dws[slot], dsems[slot])

    def fire_gather(j, b):
      pltpu.async_copy(h_hbm.at[src_v.at[j]], bufs[b], gsems[b])

    pltpu.sync_copy(ei_hbm.at[0, wid], src_v)
    fetch_dst(0, 0)
    fire_gather(0, 0)
    fire_gather(1, 1)
    pltpu.sync_copy(z_hbm, agg_s.at[pl.ds(s * ROWS_PER_TILE, ROWS_PER_TILE)])
    plsc.subcore_barrier()

    def process_window(w, slot, fire_ks, drain=False):
      pltpu.make_async_copy(
          ei_hbm.at[1, wid, pl.ds(0, W)], dws[slot], dsems[slot]).wait()
      for k_ in range(W):
        j = w * W + k_
        b = k_ % 2
        pltpu.make_async_copy(
            h_hbm.at[src_v.at[j]], bufs[b], gsems[b]).wait()
        if drain and k_ not in fire_ks:
          pltpu.async_copy(bufs[b], agg_s.at[dws[slot].at[k_]],
                           dsems[slot], add=True)
        else:
          pltpu.sync_copy(bufs[b], agg_s.at[dws[slot].at[k_]], add=True)
        if k_ in fire_ks:
          fire_gather(j + 2, b)
      if drain:
        for k_ in fire_ks[-1] + 1, W - 1:
          b = k_ % 2
          pltpu.make_async_copy(
              bufs[b], agg_s.at[dws[slot].at[k_]], dsems[slot]).wait()

    ALL = tuple(range(W))

    def body2(t, _):
      fetch_dst(2 * t + 1, 1)
      process_window(2 * t, 0, ALL)
      fetch_dst(2 * t + 2, 0)
      process_window(2 * t + 1, 1, ALL)
      return ()

    lax.fori_loop(0, NWIN // 2 - 1, body2, ())
    fetch_dst(NWIN - 1, 1)
    process_window(NWIN - 2, 0, ALL)
    process_window(NWIN - 1, 1, tuple(range(W - 2)), drain=True)

    plsc.subcore_barrier()
    pltpu.sync_copy(
        agg_s.at[pl.ds(s * ROWS_PER_TILE, ROWS_PER_TILE)],
        out_hbm.at[c, pl.ds(s * ROWS_PER_TILE, ROWS_PER_TILE)])

  return k(h, ei4, zeros_tile)


BN = 2000


def _mlp1_body(x_ref, p_ref, w1_ref, b1_ref, w2_ref, b2_ref, o_ref):
  z = x_ref[...] + p_ref[0] + p_ref[1]
  t = jnp.dot(z, w1_ref[...], preferred_element_type=jnp.float32) + b1_ref[...]
  t = jnp.maximum(t, 0.0)
  u = jnp.dot(t, w2_ref[...], preferred_element_type=jnp.float32) + b2_ref[...]
  o_ref[...] = jnp.maximum(u, 0.0)


def _mlp2_body(x_ref, p_ref, w1_ref, b1_ref, w2_ref, b2_ref, wl_ref, bl_ref,
               o_ref):
  z = x_ref[...] + p_ref[0] + p_ref[1]
  t = jnp.dot(z, w1_ref[...], preferred_element_type=jnp.float32) + b1_ref[...]
  t = jnp.maximum(t, 0.0)
  u = jnp.dot(t, w2_ref[...], preferred_element_type=jnp.float32) + b2_ref[...]
  u = jnp.maximum(u, 0.0)
  o_ref[...] = jnp.dot(u, wl_ref[...], preferred_element_type=jnp.float32) + bl_ref[...]


def _full(shape):
  return pl.BlockSpec(shape, lambda i: tuple(0 for _ in shape))


def _tc_mlp1(x, parts, W1, b1, W2, b2):
  return pl.pallas_call(
      _mlp1_body,
      grid=(N // BN,),
      in_specs=[
          pl.BlockSpec((BN, D), lambda i: (i, 0)),
          pl.BlockSpec((NC, BN, D), lambda i: (0, i, 0)),
          _full(W1.shape), _full((1, D)), _full(W2.shape), _full((1, D)),
      ],
      out_specs=pl.BlockSpec((BN, D), lambda i: (i, 0)),
      out_shape=jax.ShapeDtypeStruct((N, D), jnp.float32),
  )(x, parts, W1, b1.reshape(1, D), W2, b2.reshape(1, D))


def _tc_mlp2(h, parts, W1, b1, W2, b2, Wlin, blin):
  H2 = W1.shape[1]
  return pl.pallas_call(
      _mlp2_body,
      grid=(N // BN,),
      in_specs=[
          pl.BlockSpec((BN, D), lambda i: (i, 0)),
          pl.BlockSpec((NC, BN, D), lambda i: (0, i, 0)),
          _full(W1.shape), _full((1, H2)), _full(W2.shape), _full((1, H2)),
          _full(Wlin.shape), _full((1, D)),
      ],
      out_specs=pl.BlockSpec((BN, D), lambda i: (i, 0)),
      out_shape=jax.ShapeDtypeStruct((N, D), jnp.float32),
  )(h, parts, W1, b1.reshape(1, H2), W2, b2.reshape(1, H2),
    Wlin, blin.reshape(1, D))


def kernel(x, edge_index, W1a, b1a, W2a, b2a, W1b, b1b, W2b, b2b, Wlin, blin):
  ei4 = edge_index.astype(jnp.int32).reshape(2, NW, NCHUNK, CHUNK)
  zeros_tile = jnp.zeros((ROWS_PER_TILE, D), jnp.float32)

  p1 = _sc_agg(x, ei4, zeros_tile)
  h1 = _tc_mlp1(x, p1, W1a, b1a, W2a, b2a)
  p2 = _sc_agg(h1, ei4, zeros_tile)
  out = _tc_mlp2(h1, p2, W1b, b1b, W2b, b2b, Wlin, blin)
  return out
